# baseline (speedup 1.0000x reference)
import functools

import jax
import jax.numpy as jnp
from jax import lax
from jax.experimental import pallas as pl
from jax.experimental.pallas import tpu as pltpu
from jax.experimental.pallas import tpu_sc as plsc

D = 64
B = 4096
C = 100

NC = 2
NS = 16
NW = NC * NS
BPW = B // NW
L = 16
CG = (C + L - 1) // L
CPAD = CG * L

_mesh = plsc.VectorSubcoreMesh(core_axis_name="c", subcore_axis_name="s")
_sc_params = pltpu.CompilerParams(use_tc_tiling_on_sc=False,
                                  needs_layout_passes=False)


@functools.partial(
    pl.kernel,
    mesh=_mesh,
    out_type=(
        jax.ShapeDtypeStruct((B, D), jnp.float32),
        jax.ShapeDtypeStruct((B, D), jnp.float32),
        jax.ShapeDtypeStruct((B, D), jnp.float32),
    ),
    scratch_types=[
        pltpu.VMEM((BPW,), jnp.int32),
        pltpu.VMEM((BPW,), jnp.int32),
        pltpu.VMEM((BPW,), jnp.int32),
        pltpu.VMEM((BPW, D), jnp.float32),
        pltpu.VMEM((BPW, D), jnp.float32),
        pltpu.VMEM((BPW, D), jnp.float32),
        pltpu.SemaphoreType.DMA,
        pltpu.SemaphoreType.DMA,
        pltpu.SemaphoreType.DMA,
    ],
    compiler_params=_sc_params,
)
def _small_gathers(e_tab, r_tab, u_tab, ie, ir, iu,
                   e_out, r_out, u_out,
                   ie_v, ir_v, iu_v, e_v, r_v, u_v, se, sr, su):
    wid = lax.axis_index("s") * NC + lax.axis_index("c")
    base = wid * BPW
    pltpu.sync_copy(ie.at[pl.ds(base, BPW)], ie_v)
    pltpu.sync_copy(ir.at[pl.ds(base, BPW)], ir_v)
    pltpu.sync_copy(iu.at[pl.ds(base, BPW)], iu_v)
    ce = pltpu.async_copy(e_tab.at[ie_v], e_v, se)
    cr = pltpu.async_copy(r_tab.at[ir_v], r_v, sr)
    cu = pltpu.async_copy(u_tab.at[iu_v], u_v, su)
    ce.wait()
    cr.wait()
    cu.wait()
    pltpu.sync_copy(e_v, e_out.at[pl.ds(base, BPW)])
    pltpu.sync_copy(r_v, r_out.at[pl.ds(base, BPW)])
    pltpu.sync_copy(u_v, u_out.at[pl.ds(base, BPW)])


def _mlp_body(e_ref, r_ref, u_ref, w1_ref, b1_ref, w2_ref, b2_ref, s_ref):
    w1 = w1_ref[...]
    dn = (((1,), (1,)), ((), ()))
    h = lax.dot_general(e_ref[...], w1[:, :D], dn,
                        preferred_element_type=jnp.float32,
                        precision=lax.Precision.HIGHEST)
    h = h + lax.dot_general(r_ref[...], w1[:, D:], dn,
                            preferred_element_type=jnp.float32,
                            precision=lax.Precision.HIGHEST)
    h = jnp.maximum(h + b1_ref[...], 0.0)
    q = lax.dot_general(h, w2_ref[...], dn,
                        preferred_element_type=jnp.float32,
                        precision=lax.Precision.HIGHEST)
    s_ref[...] = q + b2_ref[...] + u_ref[...]


_MLP_BLK = B // 4

_mlp = pl.pallas_call(
    _mlp_body,
    grid=(4,),
    in_specs=[
        pl.BlockSpec((_MLP_BLK, D), lambda i: (i, 0)),
        pl.BlockSpec((_MLP_BLK, D), lambda i: (i, 0)),
        pl.BlockSpec((_MLP_BLK, D), lambda i: (i, 0)),
        pl.BlockSpec((D, 2 * D), lambda i: (0, 0)),
        pl.BlockSpec((1, D), lambda i: (0, 0)),
        pl.BlockSpec((D, D), lambda i: (0, 0)),
        pl.BlockSpec((1, D), lambda i: (0, 0)),
    ],
    out_specs=pl.BlockSpec((_MLP_BLK, D), lambda i: (i, 0)),
    out_shape=jax.ShapeDtypeStruct((B, D), jnp.float32),
)


@functools.partial(
    pl.kernel,
    mesh=_mesh,
    out_type=jax.ShapeDtypeStruct((B, CPAD), jnp.float32),
    scratch_types=[
        pltpu.VMEM((BPW, C), jnp.int32),
        pltpu.VMEM((BPW, D), jnp.float32),
        pltpu.VMEM((C, D), jnp.float32),
        pltpu.VMEM((C, D), jnp.float32),
        pltpu.VMEM((BPW, CPAD), jnp.float32),
        pltpu.SemaphoreType.DMA,
        pltpu.SemaphoreType.DMA,
    ],
    compiler_params=_sc_params,
)
def _fused_gather_dot(tab, aidx, s, out,
                      aidx_v, s_v, rows0, rows1, out_v, sem0, sem1):
    wid = lax.axis_index("s") * NC + lax.axis_index("c")
    base = wid * BPW
    pltpu.sync_copy(aidx.at[pl.ds(base, BPW)], aidx_v)
    pltpu.sync_copy(s.at[pl.ds(base, BPW)], s_v)

    lanes = lax.iota(jnp.int32, L)
    cand64 = [jnp.minimum(lanes + g * L, C - 1) * D for g in range(CG)]
    zero16 = jnp.zeros((L,), jnp.int32)
    NK = D // L

    def compute(b, rows):
        schunks = [s_v[b, pl.ds(L * k, L)] for k in range(NK)]

        def dbody(dd, accs):
            ddvec = jnp.full((L,), dd, jnp.int32)
            new = list(accs)
            for k in range(NK):
                sd = schunks[k].at[ddvec].get(mode="promise_in_bounds")
                col = jnp.full((L,), dd + L * k, jnp.int32)
                for g in range(CG):
                    v = plsc.load_gather(rows, [zero16, cand64[g] + col])
                    new[g] = new[g] + v * sd
            return tuple(new)

        accs = lax.fori_loop(
            0, L, dbody, tuple(jnp.zeros((L,), jnp.float32) for _ in range(CG)))
        for g in range(CG):
            out_v[b, pl.ds(g * L, L)] = accs[g]

    pltpu.async_copy(tab.at[aidx_v.at[0]], rows0, sem0)

    nhalf = BPW // 2

    def body(i, carry):
        b0 = 2 * i
        pltpu.async_copy(tab.at[aidx_v.at[b0 + 1]], rows1, sem1)
        pltpu.make_async_copy(tab.at[aidx_v.at[b0]], rows0, sem0).wait()
        compute(b0, rows0)

        @pl.when(i + 1 < nhalf)
        def _():
            pltpu.async_copy(tab.at[aidx_v.at[b0 + 2]], rows0, sem0)

        pltpu.make_async_copy(tab.at[aidx_v.at[b0 + 1]], rows1, sem1).wait()
        compute(b0 + 1, rows1)
        return carry

    lax.fori_loop(0, nhalf, body, 0)
    pltpu.sync_copy(out_v, out.at[pl.ds(base, BPW)])


def kernel(data, e_table, r_table, u_table, W1, b1, W2, b2):
    data = data.astype(jnp.int32)
    ie = data[:, 0, 0]
    ir = data[:, 0, 1]
    iu = data[:, 0, 2]
    ia = data[:, :, 3]
    e_emb, r_emb, u_emb = _small_gathers(e_table, r_table, u_table, ie, ir, iu)
    s = _mlp(e_emb, r_emb, u_emb, W1, b1.reshape(1, D), W2, b2.reshape(1, D))
    out_full = _fused_gather_dot(e_table, ia, s)
    return out_full[:, :C]
